# baseline (speedup 1.0000x reference)
import functools

import jax
import jax.numpy as jnp
from jax import lax
from jax.experimental import pallas as pl
from jax.experimental.pallas import tpu as pltpu
from jax.experimental.pallas import tpu_sc as plsc

_D = 32
_C = 512
_NB = 2
_L = 16
_TR, _TC = 8, 128
_NR = _D // _TR
_NC = _C // _TC


@functools.lru_cache(maxsize=None)
def _make_gather(B, S, V):
    info = plsc.get_sparse_core_info()
    NC, NS = info.num_cores, info.num_subcores
    NW = NC * NS
    assert B % (_C * NW) == 0 and B // _C // NW == 1
    assert S % _NB == 0

    mesh = plsc.VectorSubcoreMesh(core_axis_name="c", subcore_axis_name="s")

    @functools.partial(
        pl.kernel,
        out_type=jax.ShapeDtypeStruct((S, _NR, B // _TC, _TR, _TC), jnp.float32),
        mesh=mesh,
        scratch_types=[
            [pltpu.VMEM((_C,), jnp.int32) for _ in range(_NB)],
            [pltpu.VMEM((_C, _D), jnp.float32) for _ in range(_NB)],
            [pltpu.VMEM((_D, _C + 1), jnp.float32) for _ in range(_NB)],
            [pltpu.VMEM((_NR, _NC, _TR, _TC), jnp.float32) for _ in range(_NB)],
            [pltpu.SemaphoreType.DMA for _ in range(_NB)],
            [pltpu.SemaphoreType.DMA for _ in range(_NB)],
        ],
        compiler_params=pltpu.CompilerParams(
            use_tc_tiling_on_sc=False,
            needs_layout_passes=False,
            skip_device_barrier=True,
        ),
    )
    def gather_kernel(
        table_hbm, idx_hbm, out_hbm, idx_v, rows_v, rowsP_v, rowsT_v, sem_g, sem_w
    ):
        wid = lax.axis_index("s") * NC + lax.axis_index("c")
        b0 = wid * _C
        c0 = wid * _NC
        lane = lax.iota(jnp.int32, _L)

        def load_idx_and_gather(s, b):
            pltpu.sync_copy(idx_hbm.at[s, pl.ds(b0, _C)], idx_v[b])
            pltpu.async_copy(table_hbm.at[idx_v[b]], rows_v[b], sem_g[b])

        def transpose(b):
            @pl.loop(0, _C, unroll=16)
            def _bb(bb):
                col = jnp.full((_L,), bb, jnp.int32)
                for h in range(_D // _L):
                    v = rows_v[b][bb, pl.ds(h * _L, _L)]
                    plsc.store_scatter(rowsP_v[b], [h * _L + lane, col], v)

            @pl.loop(0, _TC // _L)
            def _k(k):
                for r in range(_NR):
                    for dd in range(_TR):
                        d = r * _TR + dd
                        for cp in range(_NC):
                            v = rowsP_v[b][d, pl.ds(cp * _TC + k * _L, _L)]
                            rowsT_v[b][r, cp, dd, pl.ds(k * _L, _L)] = v

        def issue_wb(s, b):
            for r in range(_NR):
                for cp in range(_NC):
                    pltpu.async_copy(
                        rowsT_v[b].at[r, cp], out_hbm.at[s, r, c0 + cp], sem_w[b]
                    )

        def wait_gather(b):
            pltpu.make_async_copy(
                table_hbm.at[pl.ds(0, _C)], rows_v[b], sem_g[b]
            ).wait()

        def wait_wb(b):
            for _ in range(_NR * _NC):
                pltpu.make_async_copy(
                    rowsT_v[b].at[0, 0], out_hbm.at[0, 0, 0], sem_w[b]
                ).wait()

        def process(s, b, next_s=None):
            wait_gather(b)
            transpose(b)
            issue_wb(s, b)
            if next_s is not None:
                load_idx_and_gather(next_s, b)
            wait_wb(b)

        for b in range(_NB):
            load_idx_and_gather(b, b)

        @pl.loop(0, S // _NB - 1)
        def _outer(o):
            for b in range(_NB):
                s = o * _NB + b
                process(s, b, next_s=s + _NB)

        for b in range(_NB):
            process(S - _NB + b, b)

    return gather_kernel


def kernel(token_ids, weights):
    B0, S = token_ids.shape
    idx_t = token_ids.T.astype(jnp.int32)
    out5 = _make_gather(B0, S, weights.shape[0])(weights, idx_t)
    return out5.transpose(2, 4, 0, 1, 3).reshape(B0, S, _D)


# speedup vs baseline: 1.0146x; 1.0146x over previous
import functools

import jax
import jax.numpy as jnp
from jax import lax
from jax.experimental import pallas as pl
from jax.experimental.pallas import tpu as pltpu
from jax.experimental.pallas import tpu_sc as plsc

_D = 32
_C = 512
_NB = 2
_L = 16
_TR, _TC = 8, 128
_NR = _D // _TR
_NC = _C // _TC


@functools.lru_cache(maxsize=None)
def _make_gather(B, S, V):
    info = plsc.get_sparse_core_info()
    NC, NS = info.num_cores, info.num_subcores
    NW = NC * NS
    assert B % (_C * NW) == 0 and B // _C // NW == 1
    assert S % _NB == 0

    mesh = plsc.VectorSubcoreMesh(core_axis_name="c", subcore_axis_name="s")

    @functools.partial(
        pl.kernel,
        out_type=jax.ShapeDtypeStruct((S, _NR, B // _TC, _TR, _TC), jnp.float32),
        mesh=mesh,
        scratch_types=[
            pltpu.VMEM((S, _C), jnp.int32),
            [pltpu.VMEM((_C, _D), jnp.float32) for _ in range(_NB)],
            pltpu.VMEM((_D, _C + 1), jnp.float32),
            [pltpu.VMEM((_NR, _NC, _TR, _TC), jnp.float32) for _ in range(_NB)],
            [pltpu.SemaphoreType.DMA for _ in range(_NB)],
            [pltpu.SemaphoreType.DMA for _ in range(_NB)],
        ],
        compiler_params=pltpu.CompilerParams(
            use_tc_tiling_on_sc=False,
            needs_layout_passes=False,
            skip_device_barrier=True,
        ),
    )
    def gather_kernel(
        table_hbm, idx_hbm, out_hbm, idx_v, rows_v, rowsP_v, rowsT_v, sem_g, sem_w
    ):
        wid = lax.axis_index("s") * NC + lax.axis_index("c")
        b0 = wid * _C
        c0 = wid * _NC
        lane = lax.iota(jnp.int32, _L)

        pltpu.sync_copy(idx_hbm.at[:, pl.ds(b0, _C)], idx_v)

        def load_idx_and_gather(s, b):
            pltpu.async_copy(table_hbm.at[idx_v.at[s]], rows_v[b], sem_g[b])

        def transpose(b):
            @pl.loop(0, _C, unroll=8)
            def _bb(bb):
                col = jnp.full((_L,), bb, jnp.int32)
                for h in range(_D // _L):
                    v = rows_v[b][bb, pl.ds(h * _L, _L)]
                    plsc.store_scatter(rowsP_v, [h * _L + lane, col], v)

            @pl.loop(0, _TC // _L)
            def _k(k):
                for r in range(_NR):
                    for dd in range(_TR):
                        d = r * _TR + dd
                        for cp in range(_NC):
                            v = rowsP_v[d, pl.ds(cp * _TC + k * _L, _L)]
                            rowsT_v[b][r, cp, dd, pl.ds(k * _L, _L)] = v

        def issue_wb(s, b):
            for r in range(_NR):
                for cp in range(_NC):
                    pltpu.async_copy(
                        rowsT_v[b].at[r, cp], out_hbm.at[s, r, c0 + cp], sem_w[b]
                    )

        def wait_gather(b):
            pltpu.make_async_copy(
                table_hbm.at[pl.ds(0, _C)], rows_v[b], sem_g[b]
            ).wait()

        def wait_wb(b):
            for _ in range(_NR * _NC):
                pltpu.make_async_copy(
                    rowsT_v[b].at[0, 0], out_hbm.at[0, 0, 0], sem_w[b]
                ).wait()

        def process(s, b, next_s=None):
            wait_gather(b)
            transpose(b)
            issue_wb(s, b)
            if next_s is not None:
                load_idx_and_gather(next_s, b)
            wait_wb(b)

        for b in range(_NB):
            load_idx_and_gather(b, b)

        @pl.loop(0, S // _NB - 1)
        def _outer(o):
            for b in range(_NB):
                s = o * _NB + b
                process(s, b, next_s=s + _NB)

        for b in range(_NB):
            process(S - _NB + b, b)

    return gather_kernel


def kernel(token_ids, weights):
    B0, S = token_ids.shape
    idx_t = token_ids.T.astype(jnp.int32)
    out5 = _make_gather(B0, S, weights.shape[0])(weights, idx_t)
    return out5.transpose(2, 4, 0, 1, 3).reshape(B0, S, _D)


# speedup vs baseline: 1.0155x; 1.0009x over previous
import functools

import jax
import jax.numpy as jnp
from jax import lax
from jax.experimental import pallas as pl
from jax.experimental.pallas import tpu as pltpu
from jax.experimental.pallas import tpu_sc as plsc

_D = 32
_C = 512
_NB = 2
_L = 16
_TR, _TC = 8, 128
_NR = _D // _TR
_NC = _C // _TC


@functools.lru_cache(maxsize=None)
def _make_gather(B, S, V):
    info = plsc.get_sparse_core_info()
    NC, NS = info.num_cores, info.num_subcores
    NW = NC * NS
    assert B % (_C * NW) == 0 and B // _C // NW == 1
    assert S % _NB == 0

    mesh = plsc.VectorSubcoreMesh(core_axis_name="c", subcore_axis_name="s")

    @functools.partial(
        pl.kernel,
        out_type=jax.ShapeDtypeStruct((S, _NR, B // _TC, _TR, _TC), jnp.float32),
        mesh=mesh,
        scratch_types=[
            pltpu.VMEM((S, _C), jnp.int32),
            [pltpu.VMEM((_C, _D), jnp.float32) for _ in range(_NB)],
            pltpu.VMEM((_D, _C + 1), jnp.float32),
            [pltpu.VMEM((_NR, _NC, _TR, _TC), jnp.float32) for _ in range(_NB)],
            [pltpu.SemaphoreType.DMA for _ in range(_NB)],
            [pltpu.SemaphoreType.DMA for _ in range(_NB)],
        ],
        compiler_params=pltpu.CompilerParams(
            use_tc_tiling_on_sc=False,
            needs_layout_passes=False,
        ),
    )
    def gather_kernel(
        table_hbm, idx_hbm, out_hbm, idx_v, rows_v, rowsP_v, rowsT_v, sem_g, sem_w
    ):
        wid = lax.axis_index("s") * NC + lax.axis_index("c")
        b0 = wid * _C
        c0 = wid * _NC
        lane = lax.iota(jnp.int32, _L)

        pltpu.sync_copy(idx_hbm.at[:, pl.ds(b0, _C)], idx_v)

        def load_idx_and_gather(s, b):
            pltpu.async_copy(table_hbm.at[idx_v.at[s]], rows_v[b], sem_g[b])

        def transpose(b):
            @pl.loop(0, _C, unroll=8)
            def _bb(bb):
                col = jnp.full((_L,), bb, jnp.int32)
                for h in range(_D // _L):
                    v = rows_v[b][bb, pl.ds(h * _L, _L)]
                    plsc.store_scatter(rowsP_v, [h * _L + lane, col], v)

            @pl.loop(0, _TC // _L)
            def _k(k):
                for r in range(_NR):
                    for dd in range(_TR):
                        d = r * _TR + dd
                        for cp in range(_NC):
                            v = rowsP_v[d, pl.ds(cp * _TC + k * _L, _L)]
                            rowsT_v[b][r, cp, dd, pl.ds(k * _L, _L)] = v

        def issue_wb(s, b):
            for r in range(_NR):
                for cp in range(_NC):
                    pltpu.async_copy(
                        rowsT_v[b].at[r, cp], out_hbm.at[s, r, c0 + cp], sem_w[b]
                    )

        def wait_gather(b):
            pltpu.make_async_copy(
                table_hbm.at[pl.ds(0, _C)], rows_v[b], sem_g[b]
            ).wait()

        def wait_wb(b):
            for _ in range(_NR * _NC):
                pltpu.make_async_copy(
                    rowsT_v[b].at[0, 0], out_hbm.at[0, 0, 0], sem_w[b]
                ).wait()

        def process(s, b, next_s=None):
            wait_gather(b)
            transpose(b)
            issue_wb(s, b)
            if next_s is not None:
                load_idx_and_gather(next_s, b)
            wait_wb(b)

        for b in range(_NB):
            load_idx_and_gather(b, b)

        @pl.loop(0, S // _NB - 1)
        def _outer(o):
            for b in range(_NB):
                s = o * _NB + b
                process(s, b, next_s=s + _NB)

        for b in range(_NB):
            process(S - _NB + b, b)

    return gather_kernel


def kernel(token_ids, weights):
    B0, S = token_ids.shape
    idx_t = token_ids.T.astype(jnp.int32)
    out5 = _make_gather(B0, S, weights.shape[0])(weights, idx_t)
    return out5.transpose(2, 4, 0, 1, 3).reshape(B0, S, _D)
